# baseline (speedup 1.0000x reference)
import functools

import jax
import jax.numpy as jnp
from jax import lax
from jax.experimental import pallas as pl
from jax.experimental.pallas import tpu as pltpu
from jax.experimental.pallas import tpu_sc as plsc

_DIM = 128

_MESH = plsc.VectorSubcoreMesh(core_axis_name="c", subcore_axis_name="s")


@functools.partial(
    pl.kernel,
    mesh=_MESH,
    out_type=jax.ShapeDtypeStruct((1, _DIM), jnp.float32),
    scratch_types=[
        pltpu.VMEM((1,), jnp.int32),
        pltpu.VMEM((1, _DIM), jnp.float32),
        pltpu.SemaphoreType.DMA,
    ],
)
def _gather_row(table_hbm, idx_hbm, out_hbm, idx_v, row_v, sem):
    cid = lax.axis_index("c")
    sid = lax.axis_index("s")

    @pl.when(jnp.logical_and(cid == 0, sid == 0))
    def _():
        pltpu.sync_copy(idx_hbm, idx_v)
        pltpu.async_copy(table_hbm.at[idx_v], row_v, sem).wait()
        pltpu.sync_copy(row_v, out_hbm)


def kernel(table, value):
    idx = jnp.reshape(value, (1,)).astype(jnp.int32)
    return jnp.reshape(_gather_row(table, idx), (_DIM,))


# speedup vs baseline: 1.0825x; 1.0825x over previous
import functools

import jax
import jax.numpy as jnp
from jax import lax
from jax.experimental import pallas as pl
from jax.experimental.pallas import tpu as pltpu
from jax.experimental.pallas import tpu_sc as plsc

_DIM = 128

_MESH = plsc.ScalarSubcoreMesh(
    axis_name="core", num_cores=plsc.get_sparse_core_info().num_cores
)


@functools.partial(
    pl.kernel,
    mesh=_MESH,
    out_type=jax.ShapeDtypeStruct((_DIM,), jnp.float32),
    scratch_types=[
        pltpu.SMEM((1,), jnp.int32),
        pltpu.SemaphoreType.DMA,
    ],
)
def _gather_row(table_hbm, idx_hbm, out_hbm, idx_s, sem):
    @pl.when(lax.axis_index("core") == 0)
    def _():
        pltpu.async_copy(idx_hbm, idx_s, sem).wait()
        idx = idx_s[0]
        pltpu.async_copy(table_hbm.at[idx], out_hbm, sem).wait()


def kernel(table, value):
    idx = jnp.reshape(value, (1,)).astype(jnp.int32)
    return _gather_row(table, idx)
